# Initial kernel scaffold; baseline (speedup 1.0000x reference)
import jax
import jax.numpy as jnp
from jax.experimental import pallas as pl


def kernel(x, unnormalized_widths, unnormalized_heights, unnormalized_derivatives, unnormalized_lambdas):
    raise NotImplementedError("write your pallas kernel here")


# speedup vs baseline: 108.4380x; 108.4380x over previous
import functools
import jax
import jax.numpy as jnp
from jax.experimental import pallas as pl
from jax.experimental.pallas import tpu as pltpu

D = 2048
K = 8
BOUND = 3.0
MIN_BW = 1e-3
MIN_BH = 1e-3
MIN_D = 1e-3
MIN_L = 0.025

NSUB = 2 * K


def _softmax0(v):
    m = jnp.max(v, axis=0, keepdims=True)
    e = jnp.exp(v - m)
    return e / jnp.sum(e, axis=0, keepdims=True)


def _knot_rows(frac):
    acc = frac[0:1, :]
    cs = [acc]
    for k in range(1, K):
        acc = acc + frac[k : k + 1, :]
        cs.append(acc)
    ones = jnp.ones_like(frac[0:1, :])
    rows = [jnp.full_like(frac[0:1, :], -BOUND)]
    for k in range(K - 1):
        rows.append(2.0 * BOUND * cs[k] - BOUND)
    rows.append(BOUND * ones)
    kn = jnp.concatenate(rows, axis=0)
    lengths = kn[1:, :] - kn[:-1, :]
    return lengths, kn


def _prep_kernel(uw_ref, uh_ref, ud_ref, ul_ref, a_ref, b_ref, g_ref, e_ref, bnd_ref):
    uw = uw_ref[...]
    uh = uh_ref[...]
    ud = ud_ref[0 : K - 1, :]
    ul = ul_ref[...]

    w = MIN_BW + (1.0 - MIN_BW * K) * _softmax0(uw)
    h = MIN_BH + (1.0 - MIN_BH * K) * _softmax0(uh)
    wf, cw = _knot_rows(w)
    hf, ch = _knot_rows(h)

    sp = jnp.maximum(ud, 0.0) + jnp.log(1.0 + jnp.exp(-jnp.abs(ud)))
    dmid = MIN_D + sp
    dend = jnp.full_like(dmid[0:1, :], 1.0 - MIN_D)
    dfull = jnp.concatenate([dend, dmid, dend], axis=0)

    lam = (1.0 - 2.0 * MIN_L) / (1.0 + jnp.exp(-ul)) + MIN_L

    d0 = dfull[:-1, :]
    d1 = dfull[1:, :]
    delta = hf / wf
    wb = jnp.sqrt(d0 / d1)
    wc = (lam * d0 + (1.0 - lam) * wb * d1) / delta
    ya = ch[:-1, :]
    yb = ch[:-1, :] + hf
    yc = ((1.0 - lam) * ya + lam * wb * yb) / ((1.0 - lam) + lam * wb)
    iw = 1.0 / wf
    cwl = cw[:-1, :]
    t0 = -cwl * iw
    wcyc = wc * yc
    wbyb = wb * yb

    a1 = ya * lam + t0 * (wcyc - ya)
    b1 = iw * (wcyc - ya)
    g1 = lam + t0 * (wc - 1.0)
    e1 = iw * (wc - 1.0)
    a2 = wcyc - lam * wbyb + t0 * (wbyb - wcyc)
    b2 = iw * (wbyb - wcyc)
    g2 = wc - lam * wb + t0 * (wb - wc)
    e2 = iw * (wb - wc)

    def ilv(p, q):
        rows = []
        for k in range(K):
            rows.append(p[k : k + 1, :])
            rows.append(q[k : k + 1, :])
        return jnp.concatenate(rows, axis=0)

    a_ref[...] = ilv(a1, a2)
    b_ref[...] = ilv(b1, b2)
    g_ref[...] = ilv(g1, g2)
    e_ref[...] = ilv(e1, e2)
    split = cwl + lam * wf
    bnd = ilv(split, cw[1:, :])
    big = jnp.full_like(split[0:1, :], 3.4e38)
    bnd_ref[...] = jnp.concatenate([bnd[:-1, :], big], axis=0)


def _prep_tables(uw, uh, ud, ul):
    uwT = uw.T
    uhT = uh.T
    udT = jnp.pad(ud.T, ((0, 1), (0, 0)))
    ulT = ul.T
    shp = jax.ShapeDtypeStruct((NSUB, D), jnp.float32)
    return pl.pallas_call(
        _prep_kernel,
        out_shape=[shp] * 5,
    )(uwT, uhT, udT, ulT)


def _tc_main_kernel(x_ref, a_ref, b_ref, g_ref, e_ref, bnd_ref, o_ref):
    x = x_ref[...]
    xc = jnp.clip(x, -BOUND, BOUND)
    shape = x.shape
    bc = lambda r: jnp.broadcast_to(r, shape)
    a = bc(a_ref[0:1, :])
    b = bc(b_ref[0:1, :])
    g = bc(g_ref[0:1, :])
    e = bc(e_ref[0:1, :])
    for j in range(1, NSUB):
        m = xc >= bnd_ref[j - 1 : j, :]
        a = jnp.where(m, bc(a_ref[j : j + 1, :]), a)
        b = jnp.where(m, bc(b_ref[j : j + 1, :]), b)
        g = jnp.where(m, bc(g_ref[j : j + 1, :]), g)
        e = jnp.where(m, bc(e_ref[j : j + 1, :]), e)
    out = (a + b * xc) / (g + e * xc)
    inside = (x >= -BOUND) & (x <= BOUND)
    o_ref[...] = jnp.where(inside, out, x)


def kernel(x, unnormalized_widths, unnormalized_heights, unnormalized_derivatives,
           unnormalized_lambdas):
    a, b, g, e, bnd = _prep_tables(unnormalized_widths, unnormalized_heights,
                                   unnormalized_derivatives, unnormalized_lambdas)
    n = x.shape[0] * x.shape[1]
    x2 = x.reshape(n, D)
    br = 256
    tab_spec = pl.BlockSpec((NSUB, D), lambda i: (0, 0))
    out = pl.pallas_call(
        _tc_main_kernel,
        grid=(n // br,),
        in_specs=[
            pl.BlockSpec((br, D), lambda i: (i, 0)),
            tab_spec, tab_spec, tab_spec, tab_spec, tab_spec,
        ],
        out_specs=pl.BlockSpec((br, D), lambda i: (i, 0)),
        out_shape=jax.ShapeDtypeStruct((n, D), jnp.float32),
    )(x2, a, b, g, e, bnd)
    return out.reshape(x.shape)
